# baseline (speedup 1.0000x reference)
import functools

import jax
import jax.numpy as jnp
from jax import lax
from jax.experimental import pallas as pl
from jax.experimental.pallas import tpu as pltpu
from jax.experimental.pallas import tpu_sc as plsc

ARTICLE_NUM = 100000
USER_NUM = 100000
MAX_HIST = 50
MAX_TOPIC = 3
BATCH = 4096

_NC = 2
_NS = 16
_NW = _NC * _NS
_BPW = BATCH // _NW
_HPW = _BPW * MAX_HIST
_TPW = _BPW * MAX_TOPIC
_NCHUNK = _HPW // _BPW
_GROUP = 10
_WINDOWS = (0, 16, 32, MAX_HIST - 16)

_LN2 = 0.6931471805599453


def _log1p16(x):
    one = jnp.float32(1.0)
    u = one + x
    ub = lax.bitcast_convert_type(u, jnp.int32)
    e = lax.shift_right_logical(ub, 23) - 127
    mb = (ub & 0x7FFFFF) | 0x3F800000
    m = lax.bitcast_convert_type(mb, jnp.float32)
    big = m >= jnp.float32(1.4142135623730951)
    m = jnp.where(big, m * jnp.float32(0.5), m)
    e = e + jnp.where(big, 1, 0)
    f = m - one
    s = f / (jnp.float32(2.0) + f)
    z = s * s
    p = one + z * (jnp.float32(1 / 3) + z * (jnp.float32(1 / 5)
        + z * (jnp.float32(1 / 7) + z * jnp.float32(1 / 9))))
    ln_u = e.astype(jnp.float32) * jnp.float32(_LN2) + jnp.float32(2.0) * s * p
    return ln_u + (x - (u - one)) / u


def _pipelined_flood(table_h, idx_v, dst_v, sem):
    prev = None
    for g in range(_NCHUNK // _GROUP):
        cps = [
            pltpu.async_copy(
                table_h.at[idx_v.at[pl.ds((g * _GROUP + b) * _BPW, _BPW)]],
                dst_v.at[pl.ds((g * _GROUP + b) * _BPW, _BPW)], sem)
            for b in range(_GROUP)
        ]
        if prev is not None:
            for cp in prev:
                cp.wait()
        prev = cps
    for cp in prev:
        cp.wait()


def _sc_body(uid_h, tid_h, time_h, pub_h, typ_h, topf_h, cat_h, sub_h, sen_h,
             histf_h,
             o_hist, o_hcat, o_typ, o_top, o_cat, o_sub, o_sen, o_emb,
             uid_v, tid_v, time_v, hidx_v, hist_v, hcat_v, tidx_v, typ_v,
             top_v, cat_v, sub_v, sen_v, pub_v, hist2_v, hcat2_v, top2_v,
             emb2_v,
             sem_typ, sem_cat, sem_sub, sem_sen, sem_pub, sem_hist, sem_top):
    wid = lax.axis_index("s") * _NC + lax.axis_index("c")
    base = wid * _BPW

    pltpu.sync_copy(uid_h.at[pl.ds(base, _BPW)], uid_v)
    pltpu.sync_copy(tid_h.at[pl.ds(base, _BPW)], tid_v)
    pltpu.sync_copy(time_h.at[pl.ds(base, _BPW)], time_v)

    cp_typ = pltpu.async_copy(typ_h.at[tid_v], typ_v, sem_typ)
    cp_cat = pltpu.async_copy(cat_h.at[tid_v], cat_v, sem_cat)
    cp_sub = pltpu.async_copy(sub_h.at[tid_v], sub_v, sem_sub)
    cp_sen = pltpu.async_copy(sen_h.at[tid_v], sen_v, sem_sen)
    cp_pub = pltpu.async_copy(pub_h.at[tid_v], pub_v, sem_pub)

    lanes = lax.iota(jnp.int32, 16)

    def _hidx_row(i, carry):
        u = plsc.load_gather(uid_v, [jnp.full((16,), i, jnp.int32)])
        u50 = u * MAX_HIST
        for w in _WINDOWS:
            hidx_v[pl.ds(i * MAX_HIST + w, 16)] = u50 + (w + lanes)
        return carry

    lax.fori_loop(0, _BPW, _hidx_row, 0)

    def _tidx_vec(c, carry):
        p = c * 16 + lanes
        b = p // MAX_TOPIC
        j = p - b * MAX_TOPIC
        t = plsc.load_gather(tid_v, [b])
        tidx_v[pl.ds(c * 16, 16)] = t * MAX_TOPIC + j
        return carry

    lax.fori_loop(0, _TPW // 16, _tidx_vec, 0)

    cp_top = [
        pltpu.async_copy(topf_h.at[tidx_v.at[pl.ds(c * _BPW, _BPW)]],
                         top_v.at[pl.ds(c * _BPW, _BPW)], sem_top)
        for c in range(MAX_TOPIC)
    ]

    _pipelined_flood(histf_h, hidx_v, hist_v, sem_hist)

    hi = jnp.full((16,), ARTICLE_NUM - 1, jnp.int32)
    lo = jnp.zeros((16,), jnp.int32)

    def _hist_row(i, carry):
        for w in _WINDOWS:
            sl = pl.ds(i * MAX_HIST + w, 16)
            v = hist_v[sl]
            hidx_v[sl] = jnp.minimum(jnp.maximum(v, lo), hi)
            hist2_v[i, pl.ds(w, 16)] = v
        return carry

    lax.fori_loop(0, _BPW, _hist_row, 0)

    _pipelined_flood(cat_h, hidx_v, hcat_v, sem_hist)

    def _hcat_row(i, carry):
        for w in _WINDOWS:
            hcat2_v[i, pl.ds(w, 16)] = hcat_v[pl.ds(i * MAX_HIST + w, 16)]
        return carry

    lax.fori_loop(0, _BPW, _hcat_row, 0)

    for cp in cp_top:
        cp.wait()

    def _top_vec(c, carry):
        p = c * 16 + lanes
        b = p // MAX_TOPIC
        j = p - b * MAX_TOPIC
        plsc.store_scatter(top2_v, [b, j], top_v[pl.ds(c * 16, 16)])
        return carry

    lax.fori_loop(0, _TPW // 16, _top_vec, 0)

    cp_pub.wait()
    zero16 = jnp.zeros((16,), jnp.int32)
    for c in range(_BPW // 16):
        sl = pl.ds(c * 16, 16)
        t = time_v[sl]
        pub = pub_v[sl]
        pub = jnp.where(pub == jnp.float32(0.0), t, pub)
        x = jnp.maximum(t - pub, jnp.float32(0.0)) * jnp.float32(1.0 / 3600000.0)
        plsc.store_scatter(emb2_v, [c * 16 + lanes, zero16], _log1p16(x))

    pltpu.sync_copy(hist2_v, o_hist.at[pl.ds(base, _BPW)])
    pltpu.sync_copy(hcat2_v, o_hcat.at[pl.ds(base, _BPW)])
    pltpu.sync_copy(top2_v, o_top.at[pl.ds(base, _BPW)])
    pltpu.sync_copy(emb2_v, o_emb.at[pl.ds(base, _BPW)])
    cp_typ.wait()
    pltpu.sync_copy(typ_v, o_typ.at[pl.ds(base, _BPW)])
    cp_cat.wait()
    pltpu.sync_copy(cat_v, o_cat.at[pl.ds(base, _BPW)])
    cp_sub.wait()
    pltpu.sync_copy(sub_v, o_sub.at[pl.ds(base, _BPW)])
    cp_sen.wait()
    pltpu.sync_copy(sen_v, o_sen.at[pl.ds(base, _BPW)])


_sc_lookup = functools.partial(
    pl.kernel,
    mesh=plsc.VectorSubcoreMesh(core_axis_name="c", subcore_axis_name="s"),
    compiler_params=pltpu.CompilerParams(needs_layout_passes=False),
    out_type=(
        jax.ShapeDtypeStruct((BATCH, MAX_HIST), jnp.int32),
        jax.ShapeDtypeStruct((BATCH, MAX_HIST), jnp.int32),
        jax.ShapeDtypeStruct((BATCH,), jnp.int32),
        jax.ShapeDtypeStruct((BATCH, MAX_TOPIC), jnp.int32),
        jax.ShapeDtypeStruct((BATCH,), jnp.int32),
        jax.ShapeDtypeStruct((BATCH,), jnp.int32),
        jax.ShapeDtypeStruct((BATCH,), jnp.int32),
        jax.ShapeDtypeStruct((BATCH, 1), jnp.float32),
    ),
    scratch_types=[
        pltpu.VMEM((_BPW,), jnp.int32),
        pltpu.VMEM((_BPW,), jnp.int32),
        pltpu.VMEM((_BPW,), jnp.float32),
        pltpu.VMEM((_HPW,), jnp.int32),
        pltpu.VMEM((_HPW,), jnp.int32),
        pltpu.VMEM((_HPW,), jnp.int32),
        pltpu.VMEM((_TPW,), jnp.int32),
        pltpu.VMEM((_BPW,), jnp.int32),
        pltpu.VMEM((_TPW,), jnp.int32),
        pltpu.VMEM((_BPW,), jnp.int32),
        pltpu.VMEM((_BPW,), jnp.int32),
        pltpu.VMEM((_BPW,), jnp.int32),
        pltpu.VMEM((_BPW,), jnp.float32),
        pltpu.VMEM((_BPW, MAX_HIST), jnp.int32),
        pltpu.VMEM((_BPW, MAX_HIST), jnp.int32),
        pltpu.VMEM((_BPW, MAX_TOPIC), jnp.int32),
        pltpu.VMEM((_BPW, 1), jnp.float32),
        pltpu.SemaphoreType.DMA,
        pltpu.SemaphoreType.DMA,
        pltpu.SemaphoreType.DMA,
        pltpu.SemaphoreType.DMA,
        pltpu.SemaphoreType.DMA,
        pltpu.SemaphoreType.DMA,
        pltpu.SemaphoreType.DMA,
    ],
)(_sc_body)


def kernel(user_id, target_id, imp_time, article_to_pub_ts, article_to_type,
           article_to_topics, article_to_cat, article_to_subcat,
           article_to_senti, train_history):
    return _sc_lookup(
        user_id, target_id, imp_time, article_to_pub_ts, article_to_type,
        article_to_topics.reshape(-1), article_to_cat, article_to_subcat,
        article_to_senti, train_history.reshape(-1))


# speedup vs baseline: 6.1107x; 6.1107x over previous
import functools

import jax
import jax.numpy as jnp
from jax import lax
from jax.experimental import pallas as pl
from jax.experimental.pallas import tpu as pltpu
from jax.experimental.pallas import tpu_sc as plsc

ARTICLE_NUM = 100000
USER_NUM = 100000
MAX_HIST = 50
MAX_TOPIC = 3
BATCH = 4096

_NC = 2
_NS = 16
_NW = _NC * _NS
_BPW = BATCH // _NW
_HPW = _BPW * MAX_HIST
_TPW = _BPW * MAX_TOPIC
_NCHUNK = _HPW // _BPW
_GROUP = 10
_WINDOWS = (0, 16, 32, MAX_HIST - 16)

_LN2 = 0.6931471805599453


def _log1p16(x):
    one = jnp.float32(1.0)
    u = one + x
    ub = lax.bitcast_convert_type(u, jnp.int32)
    e = lax.shift_right_logical(ub, 23) - 127
    mb = (ub & 0x7FFFFF) | 0x3F800000
    m = lax.bitcast_convert_type(mb, jnp.float32)
    big = m >= jnp.float32(1.4142135623730951)
    m = jnp.where(big, m * jnp.float32(0.5), m)
    e = e + jnp.where(big, 1, 0)
    f = m - one
    s = f / (jnp.float32(2.0) + f)
    z = s * s
    p = one + z * (jnp.float32(1 / 3) + z * (jnp.float32(1 / 5)
        + z * (jnp.float32(1 / 7) + z * jnp.float32(1 / 9))))
    ln_u = e.astype(jnp.float32) * jnp.float32(_LN2) + jnp.float32(2.0) * s * p
    return ln_u + (x - (u - one)) / u


def _sc_body(uid_h, tid_h, time_h, pub_h, typ_h, topf_h, cat_h, sub_h, sen_h,
             histf_h,
             o_hist, o_hcat, o_typ, o_top, o_cat, o_sub, o_sen, o_emb,
             uid_v, tid_v, time_v, hidx_v, hist_v, hcat_v, tidx_v, typ_v,
             top_v, cat_v, sub_v, sen_v, pub_v, emb_v, cat_tab_v,
             sem_tab, sem_typ, sem_cat, sem_sub, sem_sen, sem_pub,
             sem_hist, sem_top):
    wid = lax.axis_index("s") * _NC + lax.axis_index("c")
    base = wid * _BPW

    pltpu.sync_copy(uid_h.at[pl.ds(base, _BPW)], uid_v)
    pltpu.sync_copy(tid_h.at[pl.ds(base, _BPW)], tid_v)
    pltpu.sync_copy(time_h.at[pl.ds(base, _BPW)], time_v)

    cp_tab = pltpu.async_copy(cat_h, cat_tab_v, sem_tab)

    cp_typ = pltpu.async_copy(typ_h.at[tid_v], typ_v, sem_typ)
    cp_cat = pltpu.async_copy(cat_h.at[tid_v], cat_v, sem_cat)
    cp_sub = pltpu.async_copy(sub_h.at[tid_v], sub_v, sem_sub)
    cp_sen = pltpu.async_copy(sen_h.at[tid_v], sen_v, sem_sen)
    cp_pub = pltpu.async_copy(pub_h.at[tid_v], pub_v, sem_pub)

    lanes = lax.iota(jnp.int32, 16)

    def _hidx_row(i, carry):
        u = plsc.load_gather(uid_v, [jnp.full((16,), i, jnp.int32)])
        u50 = u * MAX_HIST
        for w in _WINDOWS:
            hidx_v[pl.ds(i * MAX_HIST + w, 16)] = u50 + (w + lanes)
        return carry

    lax.fori_loop(0, _BPW, _hidx_row, 0)

    def _tidx_vec(c, carry):
        p = c * 16 + lanes
        b = p // MAX_TOPIC
        j = p - b * MAX_TOPIC
        t = plsc.load_gather(tid_v, [b])
        tidx_v[pl.ds(c * 16, 16)] = t * MAX_TOPIC + j
        return carry

    lax.fori_loop(0, _TPW // 16, _tidx_vec, 0)

    cp_top = [
        pltpu.async_copy(topf_h.at[tidx_v.at[pl.ds(c * _BPW, _BPW)]],
                         top_v.at[pl.ds(c * _BPW, _BPW)], sem_top)
        for c in range(MAX_TOPIC)
    ]

    prev = None
    for g in range(_NCHUNK // _GROUP):
        cps = [
            pltpu.async_copy(
                histf_h.at[hidx_v.at[pl.ds((g * _GROUP + b) * _BPW, _BPW)]],
                hist_v.at[pl.ds((g * _GROUP + b) * _BPW, _BPW)], sem_hist)
            for b in range(_GROUP)
        ]
        if prev is not None:
            for cp in prev:
                cp.wait()
        prev = cps
    for cp in prev:
        cp.wait()

    cp_tab.wait()
    hi = jnp.full((16,), ARTICLE_NUM - 1, jnp.int32)
    lo = jnp.zeros((16,), jnp.int32)

    def _hcat_vec(c, carry):
        idx = hist_v[pl.ds(c * 16, 16)]
        idx = jnp.minimum(jnp.maximum(idx, lo), hi)
        hcat_v[pl.ds(c * 16, 16)] = plsc.load_gather(cat_tab_v, [idx])
        return carry

    lax.fori_loop(0, _HPW // 16, _hcat_vec, 0)

    cp_pub.wait()
    for c in range(_BPW // 16):
        sl = pl.ds(c * 16, 16)
        t = time_v[sl]
        pub = pub_v[sl]
        pub = jnp.where(pub == jnp.float32(0.0), t, pub)
        x = jnp.maximum(t - pub, jnp.float32(0.0)) * jnp.float32(1.0 / 3600000.0)
        emb_v[sl] = _log1p16(x)

    pltpu.sync_copy(hist_v, o_hist.at[pl.ds(base * MAX_HIST, _HPW)])
    pltpu.sync_copy(hcat_v, o_hcat.at[pl.ds(base * MAX_HIST, _HPW)])
    pltpu.sync_copy(emb_v, o_emb.at[pl.ds(base, _BPW)])
    for cp in cp_top:
        cp.wait()
    pltpu.sync_copy(top_v, o_top.at[pl.ds(base * MAX_TOPIC, _TPW)])
    cp_typ.wait()
    pltpu.sync_copy(typ_v, o_typ.at[pl.ds(base, _BPW)])
    cp_cat.wait()
    pltpu.sync_copy(cat_v, o_cat.at[pl.ds(base, _BPW)])
    cp_sub.wait()
    pltpu.sync_copy(sub_v, o_sub.at[pl.ds(base, _BPW)])
    cp_sen.wait()
    pltpu.sync_copy(sen_v, o_sen.at[pl.ds(base, _BPW)])


_sc_lookup = functools.partial(
    pl.kernel,
    mesh=plsc.VectorSubcoreMesh(core_axis_name="c", subcore_axis_name="s"),
    compiler_params=pltpu.CompilerParams(use_tc_tiling_on_sc=False,
                                         needs_layout_passes=False),
    out_type=(
        jax.ShapeDtypeStruct((BATCH * MAX_HIST,), jnp.int32),
        jax.ShapeDtypeStruct((BATCH * MAX_HIST,), jnp.int32),
        jax.ShapeDtypeStruct((BATCH,), jnp.int32),
        jax.ShapeDtypeStruct((BATCH * MAX_TOPIC,), jnp.int32),
        jax.ShapeDtypeStruct((BATCH,), jnp.int32),
        jax.ShapeDtypeStruct((BATCH,), jnp.int32),
        jax.ShapeDtypeStruct((BATCH,), jnp.int32),
        jax.ShapeDtypeStruct((BATCH,), jnp.float32),
    ),
    scratch_types=[
        pltpu.VMEM((_BPW,), jnp.int32),
        pltpu.VMEM((_BPW,), jnp.int32),
        pltpu.VMEM((_BPW,), jnp.float32),
        pltpu.VMEM((_HPW,), jnp.int32),
        pltpu.VMEM((_HPW,), jnp.int32),
        pltpu.VMEM((_HPW,), jnp.int32),
        pltpu.VMEM((_TPW,), jnp.int32),
        pltpu.VMEM((_BPW,), jnp.int32),
        pltpu.VMEM((_TPW,), jnp.int32),
        pltpu.VMEM((_BPW,), jnp.int32),
        pltpu.VMEM((_BPW,), jnp.int32),
        pltpu.VMEM((_BPW,), jnp.int32),
        pltpu.VMEM((_BPW,), jnp.float32),
        pltpu.VMEM((_BPW,), jnp.float32),
        pltpu.VMEM((ARTICLE_NUM,), jnp.int32),
        pltpu.SemaphoreType.DMA,
        pltpu.SemaphoreType.DMA,
        pltpu.SemaphoreType.DMA,
        pltpu.SemaphoreType.DMA,
        pltpu.SemaphoreType.DMA,
        pltpu.SemaphoreType.DMA,
        pltpu.SemaphoreType.DMA,
        pltpu.SemaphoreType.DMA,
    ],
)(_sc_body)


def kernel(user_id, target_id, imp_time, article_to_pub_ts, article_to_type,
           article_to_topics, article_to_cat, article_to_subcat,
           article_to_senti, train_history):
    (hist, hcat, ttyp, ttop, tcat, tsub, tsen, emb) = _sc_lookup(
        user_id, target_id, imp_time, article_to_pub_ts, article_to_type,
        article_to_topics.reshape(-1), article_to_cat, article_to_subcat,
        article_to_senti, train_history.reshape(-1))
    return (hist.reshape(BATCH, MAX_HIST),
            hcat.reshape(BATCH, MAX_HIST),
            ttyp,
            ttop.reshape(BATCH, MAX_TOPIC),
            tcat, tsub, tsen, emb.reshape(BATCH, 1))


# speedup vs baseline: 7.1381x; 1.1681x over previous
import functools

import jax
import jax.numpy as jnp
from jax import lax
from jax.experimental import pallas as pl
from jax.experimental.pallas import tpu as pltpu
from jax.experimental.pallas import tpu_sc as plsc

ARTICLE_NUM = 100000
USER_NUM = 100000
MAX_HIST = 50
MAX_TOPIC = 3
BATCH = 4096

_NC = 2
_NS = 16
_NW = _NC * _NS
_BPW = BATCH // _NW
_HPW = _BPW * MAX_HIST
_TPW = _BPW * MAX_TOPIC
_RING = 8
_BGRP = 4
_WINDOWS = (0, 16, 32, MAX_HIST - 16)

_LN2 = 0.6931471805599453


def _log1p16(x):
    one = jnp.float32(1.0)
    u = one + x
    ub = lax.bitcast_convert_type(u, jnp.int32)
    e = lax.shift_right_logical(ub, 23) - 127
    mb = (ub & 0x7FFFFF) | 0x3F800000
    m = lax.bitcast_convert_type(mb, jnp.float32)
    big = m >= jnp.float32(1.4142135623730951)
    m = jnp.where(big, m * jnp.float32(0.5), m)
    e = e + jnp.where(big, 1, 0)
    f = m - one
    s = f / (jnp.float32(2.0) + f)
    z = s * s
    p = one + z * (jnp.float32(1 / 3) + z * (jnp.float32(1 / 5)
        + z * (jnp.float32(1 / 7) + z * jnp.float32(1 / 9))))
    ln_u = e.astype(jnp.float32) * jnp.float32(_LN2) + jnp.float32(2.0) * s * p
    return ln_u + (x - (u - one)) / u


def _sc_body(uid_h, tid_h, time_h, pub_h, typ_h, topf_h, cat_h, sub_h, sen_h,
             histf_h,
             o_hist, o_hcat, o_typ, o_top, o_cat, o_sub, o_sen, o_emb,
             uid_v, tid_v, time_v, hring_v, hist_v, hcat_v, tidx_v, typ_v,
             top_v, cat_v, sub_v, sen_v, pub_v, emb_v, cat_tab_v,
             sem_tab, sem_typ, sem_cat, sem_sub, sem_sen, sem_pub,
             sem_hist, sem_top):
    wid = lax.axis_index("s") * _NC + lax.axis_index("c")
    base = wid * _BPW

    pltpu.sync_copy(uid_h.at[pl.ds(base, _BPW)], uid_v)
    pltpu.sync_copy(tid_h.at[pl.ds(base, _BPW)], tid_v)
    pltpu.sync_copy(time_h.at[pl.ds(base, _BPW)], time_v)

    cp_tab = pltpu.async_copy(cat_h, cat_tab_v, sem_tab)

    cp_typ = pltpu.async_copy(typ_h.at[tid_v], typ_v, sem_typ)
    cp_cat = pltpu.async_copy(cat_h.at[tid_v], cat_v, sem_cat)
    cp_sub = pltpu.async_copy(sub_h.at[tid_v], sub_v, sem_sub)
    cp_sen = pltpu.async_copy(sen_h.at[tid_v], sen_v, sem_sen)
    cp_pub = pltpu.async_copy(pub_h.at[tid_v], pub_v, sem_pub)

    lanes = lax.iota(jnp.int32, 16)

    def _tidx_vec(c, carry):
        p = c * 16 + lanes
        b = p // MAX_TOPIC
        j = p - b * MAX_TOPIC
        t = plsc.load_gather(tid_v, [b])
        tidx_v[pl.ds(c * 16, 16)] = t * MAX_TOPIC + j
        return carry

    lax.fori_loop(0, _TPW // 16, _tidx_vec, 0)

    cp_top = [
        pltpu.async_copy(topf_h.at[tidx_v.at[pl.ds(c * _BPW, _BPW)]],
                         top_v.at[pl.ds(c * _BPW, _BPW)], sem_top)
        for c in range(MAX_TOPIC)
    ]

    def _extract_row(b, slot, q):
        for w in _WINDOWS:
            hist_v[pl.ds(b * MAX_HIST + w, 16)] = hring_v[slot, q, pl.ds(w, 16)]

    prev_cps = prev_rows = None
    for g in range(_BPW // _BGRP):
        cps, rows = [], []
        for b in range(g * _BGRP, (g + 1) * _BGRP):
            u = jnp.max(plsc.load_gather(
                uid_v, [jnp.full((16,), b, jnp.int32)]))
            u8 = pl.multiple_of((u // 8) * 8, 8)
            slot = b % _RING
            cps.append(pltpu.async_copy(histf_h.at[pl.ds(u8, 8)],
                                        hring_v.at[slot], sem_hist))
            rows.append((b, slot, u - u8))
        if prev_cps is not None:
            for cp in prev_cps:
                cp.wait()
            for b, slot, q in prev_rows:
                _extract_row(b, slot, q)
        prev_cps, prev_rows = cps, rows
    for cp in prev_cps:
        cp.wait()
    for b, slot, q in prev_rows:
        _extract_row(b, slot, q)

    cp_tab.wait()
    hi = jnp.full((16,), ARTICLE_NUM - 1, jnp.int32)
    lo = jnp.zeros((16,), jnp.int32)

    def _hcat_vec(c, carry):
        idx = hist_v[pl.ds(c * 16, 16)]
        idx = jnp.minimum(jnp.maximum(idx, lo), hi)
        hcat_v[pl.ds(c * 16, 16)] = plsc.load_gather(cat_tab_v, [idx])
        return carry

    lax.fori_loop(0, _HPW // 16, _hcat_vec, 0)

    cp_pub.wait()
    for c in range(_BPW // 16):
        sl = pl.ds(c * 16, 16)
        t = time_v[sl]
        pub = pub_v[sl]
        pub = jnp.where(pub == jnp.float32(0.0), t, pub)
        x = jnp.maximum(t - pub, jnp.float32(0.0)) * jnp.float32(1.0 / 3600000.0)
        emb_v[sl] = _log1p16(x)

    pltpu.sync_copy(hist_v, o_hist.at[pl.ds(base * MAX_HIST, _HPW)])
    pltpu.sync_copy(hcat_v, o_hcat.at[pl.ds(base * MAX_HIST, _HPW)])
    pltpu.sync_copy(emb_v, o_emb.at[pl.ds(base, _BPW)])
    for cp in cp_top:
        cp.wait()
    pltpu.sync_copy(top_v, o_top.at[pl.ds(base * MAX_TOPIC, _TPW)])
    cp_typ.wait()
    pltpu.sync_copy(typ_v, o_typ.at[pl.ds(base, _BPW)])
    cp_cat.wait()
    pltpu.sync_copy(cat_v, o_cat.at[pl.ds(base, _BPW)])
    cp_sub.wait()
    pltpu.sync_copy(sub_v, o_sub.at[pl.ds(base, _BPW)])
    cp_sen.wait()
    pltpu.sync_copy(sen_v, o_sen.at[pl.ds(base, _BPW)])


_sc_lookup = functools.partial(
    pl.kernel,
    mesh=plsc.VectorSubcoreMesh(core_axis_name="c", subcore_axis_name="s"),
    compiler_params=pltpu.CompilerParams(needs_layout_passes=False),
    out_type=(
        jax.ShapeDtypeStruct((BATCH * MAX_HIST,), jnp.int32),
        jax.ShapeDtypeStruct((BATCH * MAX_HIST,), jnp.int32),
        jax.ShapeDtypeStruct((BATCH,), jnp.int32),
        jax.ShapeDtypeStruct((BATCH * MAX_TOPIC,), jnp.int32),
        jax.ShapeDtypeStruct((BATCH,), jnp.int32),
        jax.ShapeDtypeStruct((BATCH,), jnp.int32),
        jax.ShapeDtypeStruct((BATCH,), jnp.int32),
        jax.ShapeDtypeStruct((BATCH,), jnp.float32),
    ),
    scratch_types=[
        pltpu.VMEM((_BPW,), jnp.int32),
        pltpu.VMEM((_BPW,), jnp.int32),
        pltpu.VMEM((_BPW,), jnp.float32),
        pltpu.VMEM((_RING, 8, MAX_HIST), jnp.int32),
        pltpu.VMEM((_HPW,), jnp.int32),
        pltpu.VMEM((_HPW,), jnp.int32),
        pltpu.VMEM((_TPW,), jnp.int32),
        pltpu.VMEM((_BPW,), jnp.int32),
        pltpu.VMEM((_TPW,), jnp.int32),
        pltpu.VMEM((_BPW,), jnp.int32),
        pltpu.VMEM((_BPW,), jnp.int32),
        pltpu.VMEM((_BPW,), jnp.int32),
        pltpu.VMEM((_BPW,), jnp.float32),
        pltpu.VMEM((_BPW,), jnp.float32),
        pltpu.VMEM((ARTICLE_NUM,), jnp.int32),
        pltpu.SemaphoreType.DMA,
        pltpu.SemaphoreType.DMA,
        pltpu.SemaphoreType.DMA,
        pltpu.SemaphoreType.DMA,
        pltpu.SemaphoreType.DMA,
        pltpu.SemaphoreType.DMA,
        pltpu.SemaphoreType.DMA,
        pltpu.SemaphoreType.DMA,
    ],
)(_sc_body)


def kernel(user_id, target_id, imp_time, article_to_pub_ts, article_to_type,
           article_to_topics, article_to_cat, article_to_subcat,
           article_to_senti, train_history):
    (hist, hcat, ttyp, ttop, tcat, tsub, tsen, emb) = _sc_lookup(
        user_id, target_id, imp_time, article_to_pub_ts, article_to_type,
        article_to_topics.reshape(-1), article_to_cat, article_to_subcat,
        article_to_senti, train_history)
    return (hist.reshape(BATCH, MAX_HIST),
            hcat.reshape(BATCH, MAX_HIST),
            ttyp,
            ttop.reshape(BATCH, MAX_TOPIC),
            tcat, tsub, tsen, emb.reshape(BATCH, 1))


# speedup vs baseline: 8.3143x; 1.1648x over previous
import functools

import jax
import jax.numpy as jnp
from jax import lax
from jax.experimental import pallas as pl
from jax.experimental.pallas import tpu as pltpu
from jax.experimental.pallas import tpu_sc as plsc

ARTICLE_NUM = 100000
USER_NUM = 100000
MAX_HIST = 50
MAX_TOPIC = 3
BATCH = 4096

_NC = 2
_NS = 16
_NW = _NC * _NS
_BPW = BATCH // _NW
_HPW = _BPW * MAX_HIST
_TPW = _BPW * MAX_TOPIC
_RING = 4
_BGRP = 2
_WINDOWS = (0, 16, 32, MAX_HIST - 16)

_LN2 = 0.6931471805599453


def _log1p16(x):
    one = jnp.float32(1.0)
    u = one + x
    ub = lax.bitcast_convert_type(u, jnp.int32)
    e = lax.shift_right_logical(ub, 23) - 127
    mb = (ub & 0x7FFFFF) | 0x3F800000
    m = lax.bitcast_convert_type(mb, jnp.float32)
    big = m >= jnp.float32(1.4142135623730951)
    m = jnp.where(big, m * jnp.float32(0.5), m)
    e = e + jnp.where(big, 1, 0)
    f = m - one
    s = f / (jnp.float32(2.0) + f)
    z = s * s
    p = one + z * (jnp.float32(1 / 3) + z * (jnp.float32(1 / 5)
        + z * (jnp.float32(1 / 7) + z * jnp.float32(1 / 9))))
    ln_u = e.astype(jnp.float32) * jnp.float32(_LN2) + jnp.float32(2.0) * s * p
    return ln_u + (x - (u - one)) / u


def _sc_body(uid_h, tid_h, time_h, pub_h, typ_h, topf_h, cat_h, sub_h, sen_h,
             histf_h,
             o_hist, o_hcat, o_typ, o_top, o_cat, o_sub, o_sen, o_emb,
             uid_v, tid_v, time_v, hring_v, tring_v, hist_v, hcat_v, typ_v,
             top_v, cat_v, sub_v, sen_v, pub_v, emb_v, cat_tab_v,
             sem_tab, sem_typ, sem_cat, sem_sub, sem_sen, sem_pub,
             sem_hist, sem_top):
    wid = lax.axis_index("s") * _NC + lax.axis_index("c")
    base = wid * _BPW

    pltpu.sync_copy(uid_h.at[pl.ds(base, _BPW)], uid_v)
    pltpu.sync_copy(tid_h.at[pl.ds(base, _BPW)], tid_v)
    pltpu.sync_copy(time_h.at[pl.ds(base, _BPW)], time_v)

    cp_tab = pltpu.async_copy(cat_h, cat_tab_v, sem_tab)

    cp_typ = pltpu.async_copy(typ_h.at[tid_v], typ_v, sem_typ)
    cp_cat = pltpu.async_copy(cat_h.at[tid_v], cat_v, sem_cat)
    cp_sub = pltpu.async_copy(sub_h.at[tid_v], sub_v, sem_sub)
    cp_sen = pltpu.async_copy(sen_h.at[tid_v], sen_v, sem_sen)
    cp_pub = pltpu.async_copy(pub_h.at[tid_v], pub_v, sem_pub)

    lanes = lax.iota(jnp.int32, 16)

    lanes3 = jnp.minimum(lanes, 2)
    mask3 = lanes < MAX_TOPIC

    def _extract_row(b, slot, q, r):
        for w in _WINDOWS:
            hist_v[pl.ds(b * MAX_HIST + w, 16)] = hring_v[slot, q, pl.ds(w, 16)]
        tvals = plsc.load_gather(
            tring_v, [jnp.full((16,), slot, jnp.int32),
                      jnp.full((16,), r, jnp.int32), lanes3])
        plsc.store_scatter(top_v, [b * MAX_TOPIC + lanes], tvals, mask=mask3)

    prev_cps = prev_rows = None
    for g in range(_BPW // _BGRP):
        cps, rows = [], []
        for b in range(g * _BGRP, (g + 1) * _BGRP):
            u = jnp.max(plsc.load_gather(
                uid_v, [jnp.full((16,), b, jnp.int32)]))
            u8 = pl.multiple_of((u // 8) * 8, 8)
            t = jnp.max(plsc.load_gather(
                tid_v, [jnp.full((16,), b, jnp.int32)]))
            t8 = pl.multiple_of((t // 8) * 8, 8)
            slot = b % _RING
            cps.append(pltpu.async_copy(histf_h.at[pl.ds(u8, 8)],
                                        hring_v.at[slot], sem_hist))
            cps.append(pltpu.async_copy(topf_h.at[pl.ds(t8, 8)],
                                        tring_v.at[slot], sem_top))
            rows.append((b, slot, u - u8, t - t8))
        if prev_cps is not None:
            for cp in prev_cps:
                cp.wait()
            for b, slot, q, r in prev_rows:
                _extract_row(b, slot, q, r)
        prev_cps, prev_rows = cps, rows
    for cp in prev_cps:
        cp.wait()
    for b, slot, q, r in prev_rows:
        _extract_row(b, slot, q, r)

    cp_tab.wait()
    hi = jnp.full((16,), ARTICLE_NUM - 1, jnp.int32)
    lo = jnp.zeros((16,), jnp.int32)

    def _hcat_vec(c, carry):
        idx = hist_v[pl.ds(c * 16, 16)]
        idx = jnp.minimum(jnp.maximum(idx, lo), hi)
        hcat_v[pl.ds(c * 16, 16)] = plsc.load_gather(cat_tab_v, [idx])
        return carry

    lax.fori_loop(0, _HPW // 16, _hcat_vec, 0)

    cp_pub.wait()
    for c in range(_BPW // 16):
        sl = pl.ds(c * 16, 16)
        t = time_v[sl]
        pub = pub_v[sl]
        pub = jnp.where(pub == jnp.float32(0.0), t, pub)
        x = jnp.maximum(t - pub, jnp.float32(0.0)) * jnp.float32(1.0 / 3600000.0)
        emb_v[sl] = _log1p16(x)

    pltpu.sync_copy(hist_v, o_hist.at[pl.ds(base * MAX_HIST, _HPW)])
    pltpu.sync_copy(hcat_v, o_hcat.at[pl.ds(base * MAX_HIST, _HPW)])
    pltpu.sync_copy(emb_v, o_emb.at[pl.ds(base, _BPW)])
    pltpu.sync_copy(top_v, o_top.at[pl.ds(base * MAX_TOPIC, _TPW)])
    cp_typ.wait()
    pltpu.sync_copy(typ_v, o_typ.at[pl.ds(base, _BPW)])
    cp_cat.wait()
    pltpu.sync_copy(cat_v, o_cat.at[pl.ds(base, _BPW)])
    cp_sub.wait()
    pltpu.sync_copy(sub_v, o_sub.at[pl.ds(base, _BPW)])
    cp_sen.wait()
    pltpu.sync_copy(sen_v, o_sen.at[pl.ds(base, _BPW)])


_sc_lookup = functools.partial(
    pl.kernel,
    mesh=plsc.VectorSubcoreMesh(core_axis_name="c", subcore_axis_name="s"),
    compiler_params=pltpu.CompilerParams(needs_layout_passes=False),
    out_type=(
        jax.ShapeDtypeStruct((BATCH * MAX_HIST,), jnp.int32),
        jax.ShapeDtypeStruct((BATCH * MAX_HIST,), jnp.int32),
        jax.ShapeDtypeStruct((BATCH,), jnp.int32),
        jax.ShapeDtypeStruct((BATCH * MAX_TOPIC,), jnp.int32),
        jax.ShapeDtypeStruct((BATCH,), jnp.int32),
        jax.ShapeDtypeStruct((BATCH,), jnp.int32),
        jax.ShapeDtypeStruct((BATCH,), jnp.int32),
        jax.ShapeDtypeStruct((BATCH,), jnp.float32),
    ),
    scratch_types=[
        pltpu.VMEM((_BPW,), jnp.int32),
        pltpu.VMEM((_BPW,), jnp.int32),
        pltpu.VMEM((_BPW,), jnp.float32),
        pltpu.VMEM((_RING, 8, MAX_HIST), jnp.int32),
        pltpu.VMEM((_RING, 8, MAX_TOPIC), jnp.int32),
        pltpu.VMEM((_HPW,), jnp.int32),
        pltpu.VMEM((_HPW,), jnp.int32),
        pltpu.VMEM((_BPW,), jnp.int32),
        pltpu.VMEM((_TPW,), jnp.int32),
        pltpu.VMEM((_BPW,), jnp.int32),
        pltpu.VMEM((_BPW,), jnp.int32),
        pltpu.VMEM((_BPW,), jnp.int32),
        pltpu.VMEM((_BPW,), jnp.float32),
        pltpu.VMEM((_BPW,), jnp.float32),
        pltpu.VMEM((ARTICLE_NUM,), jnp.int32),
        pltpu.SemaphoreType.DMA,
        pltpu.SemaphoreType.DMA,
        pltpu.SemaphoreType.DMA,
        pltpu.SemaphoreType.DMA,
        pltpu.SemaphoreType.DMA,
        pltpu.SemaphoreType.DMA,
        pltpu.SemaphoreType.DMA,
        pltpu.SemaphoreType.DMA,
    ],
)(_sc_body)


def kernel(user_id, target_id, imp_time, article_to_pub_ts, article_to_type,
           article_to_topics, article_to_cat, article_to_subcat,
           article_to_senti, train_history):
    (hist, hcat, ttyp, ttop, tcat, tsub, tsen, emb) = _sc_lookup(
        user_id, target_id, imp_time, article_to_pub_ts, article_to_type,
        article_to_topics, article_to_cat, article_to_subcat,
        article_to_senti, train_history)
    return (hist.reshape(BATCH, MAX_HIST),
            hcat.reshape(BATCH, MAX_HIST),
            ttyp,
            ttop.reshape(BATCH, MAX_TOPIC),
            tcat, tsub, tsen, emb.reshape(BATCH, 1))
